# baseline (speedup 1.0000x reference)
import functools

import jax
import jax.numpy as jnp
from jax import lax
from jax.experimental import pallas as pl
from jax.experimental.pallas import tpu as pltpu
from jax.experimental.pallas import tpu_sc as plsc

_K = 16
_MAXF = 1024.0
_NC = 2
_NS = 16
_NW = _NC * _NS
_LANES = 16
_IL = 2


def _topk_sc(rows):
    R, F = rows.shape
    assert R % 8 == 0
    ngroups = R // 8
    gpw = ngroups // _NW
    extra = ngroups % _NW
    nb_max = gpw + (1 if extra else 0)
    nchunk = F // _LANES
    RB = 8

    mesh = plsc.VectorSubcoreMesh(core_axis_name="c", subcore_axis_name="s")

    @functools.partial(
        pl.kernel,
        out_type=jax.ShapeDtypeStruct((R, 2 * _K), jnp.float32),
        mesh=mesh,
        scratch_types=[
            pltpu.VMEM((2, RB, F), jnp.float32),
            pltpu.VMEM((nb_max * RB, 2 * _K), jnp.float32),
            pltpu.SemaphoreType.DMA,
            pltpu.SemaphoreType.DMA,
        ],
        compiler_params=pltpu.CompilerParams(needs_layout_passes=False),
    )
    def topk_kernel(w_hbm, cf_out, rbuf, obuf, sem0, sem1):
        wid = lax.axis_index("s") * _NC + lax.axis_index("c")
        nbatch = gpw + jnp.where(wid < extra, 1, 0)
        base = RB * (wid * gpw + jnp.minimum(wid, extra))
        iota = lax.iota(jnp.int32, _LANES)

        def fetch(b, slot, sem):
            return pltpu.make_async_copy(
                w_hbm.at[pl.ds(base + b * RB, RB)], rbuf.at[slot], sem)

        fetch(0, 0, sem0).start()

        def batch_body(b, carry):
            slot = lax.rem(b, 2)

            def even_case():
                fetch(b, 0, sem0).wait()

                @pl.when(b + 1 < nbatch)
                def _():
                    fetch(b + 1, 1, sem1).start()

            def odd_case():
                fetch(b, 1, sem1).wait()

                @pl.when(b + 1 < nbatch)
                def _():
                    fetch(b + 1, 0, sem0).start()

            lax.cond(slot == 0, even_case, odd_case)

            def row_group(rg, carry2):
                for q in range(_IL):
                    r = rg * _IL + q
                    level = []
                    for c in range(nchunk):
                        a = jnp.abs(rbuf[slot, r, pl.ds(c * _LANES, _LANES)])
                        level.append(plsc.sort_key_val(
                            a, iota + c * _LANES, descending=(c % 2 == 1)))
                    while len(level) > 1:
                        nxt = []
                        for m in range(len(level) // 2):
                            av, ai = level[2 * m]
                            bv, bi = level[2 * m + 1]
                            take = av >= bv
                            mv = jnp.where(take, av, bv)
                            mi = jnp.where(take, ai, bi)
                            nxt.append(plsc.sort_key_val(
                                mv, mi, descending=(m % 2 == 1)))
                        level = nxt
                    rv, ri = level[0]
                    slot_v = jnp.full((_LANES,), slot, jnp.int32)
                    row_v = jnp.full((_LANES,), r, jnp.int32)
                    signed = plsc.load_gather(rbuf, [slot_v, row_v, ri])
                    out_r = b * RB + r
                    obuf[out_r, pl.ds(0, _K)] = signed
                    obuf[out_r, pl.ds(_K, _K)] = (
                        ri.astype(jnp.float32) * (_MAXF / F))
                return carry2

            lax.fori_loop(0, RB // _IL, row_group, 0)
            return carry

        lax.fori_loop(0, nbatch, batch_body, 0)

        def flush_extra():
            pltpu.sync_copy(obuf.at[pl.ds(0, nb_max * RB)],
                            cf_out.at[pl.ds(base, nb_max * RB)])

        def flush_base():
            pltpu.sync_copy(obuf.at[pl.ds(0, gpw * RB)],
                            cf_out.at[pl.ds(base, gpw * RB)])

        if extra:
            lax.cond(wid < extra, flush_extra, flush_base)
        else:
            flush_base()

    return topk_kernel(rows)


_C2P = (
    0.9999994436793985,
    -19.73903437293113,
    64.9306133699045,
    -85.29597096153829,
    58.91255532441485,
    -21.28302159300549,
)


def _cos2pi(r):
    u = r - jnp.round(r)
    s = u * u
    p = jnp.float32(_C2P[5])
    for c in (_C2P[4], _C2P[3], _C2P[2], _C2P[1], _C2P[0]):
        p = p * s + jnp.float32(c)
    return p


def _render_body(x_ref, p_ref, w_ref, c3_ref, out_ref):
    i = pl.program_id(1)
    px = p_ref[0:1, :]
    py = p_ref[1:2, :]
    sx = p_ref[2:3, :]
    sy = p_ref[3:4, :]
    rot = p_ref[4:5, :]
    cr = jnp.cos(rot)
    sr = jnp.sin(rot)
    x0 = x_ref[:, 0:1]
    x1 = x_ref[:, 1:2]
    rx = x0 - px
    ry = x1 - py
    tx = cr * rx + sr * ry
    ty = cr * ry - sr * rx
    env = jnp.exp(-0.5 * ((tx * sx) ** 2 + (ty * sy) ** 2))

    wx = w_ref[0, 0:1, :] * _cos2pi(w_ref[0, 2:3, :] * tx)
    wy = w_ref[0, 1:2, :] * _cos2pi(w_ref[0, 3:4, :] * ty)
    for j in range(1, _K):
        wx = wx + w_ref[j, 0:1, :] * _cos2pi(w_ref[j, 2:3, :] * tx)
        wy = wy + w_ref[j, 1:2, :] * _cos2pi(w_ref[j, 3:4, :] * ty)
    w = env * wx * wy
    acc = lax.dot_general(w, c3_ref[...], (((1,), (0,)), ((), ())),
                          preferred_element_type=jnp.float32)

    @pl.when(i == 0)
    def _():
        out_ref[...] = jnp.zeros_like(out_ref)

    out_ref[...] += acc


def kernel(x, gaussian_colors, gaussian_positions, gaussian_scales,
           gaussian_rotations, wave_coefficients):
    N = x.shape[0]
    G, _, F = wave_coefficients.shape

    cfr = _topk_sc(wave_coefficients.reshape(G * 2, F))
    cf = cfr[:, :_K].reshape(G, 2, _K)
    fr = cfr[:, _K:].reshape(G, 2, _K)

    GBLK = 512
    Gp = ((G + GBLK - 1) // GBLK) * GBLK
    pad = Gp - G

    W = jnp.stack([cf[:, 0, :], cf[:, 1, :], fr[:, 0, :], fr[:, 1, :]],
                  axis=1)
    W = jnp.pad(W, ((0, pad), (0, 4), (0, 0))).transpose(2, 1, 0)
    P = jnp.concatenate([gaussian_positions, gaussian_scales,
                         gaussian_rotations,
                         jnp.zeros((G, 3), jnp.float32)], axis=1).T
    P = jnp.pad(P, ((0, 0), (0, pad)))
    C3 = jnp.pad(gaussian_colors, ((0, pad), (0, 0)))

    NBLK = 128
    out = pl.pallas_call(
        _render_body,
        grid=(N // NBLK, Gp // GBLK),
        in_specs=[
            pl.BlockSpec((NBLK, 2), lambda n, i: (n, 0)),
            pl.BlockSpec((8, GBLK), lambda n, i: (0, i)),
            pl.BlockSpec((_K, 8, GBLK), lambda n, i: (0, 0, i)),
            pl.BlockSpec((GBLK, 3), lambda n, i: (i, 0)),
        ],
        out_specs=pl.BlockSpec((NBLK, 3), lambda n, i: (n, 0)),
        out_shape=jax.ShapeDtypeStruct((N, 3), jnp.float32),
    )(x, P, W, C3)
    return out


# speedup vs baseline: 1.0546x; 1.0546x over previous
import functools

import jax
import jax.numpy as jnp
from jax import lax
from jax.experimental import pallas as pl
from jax.experimental.pallas import tpu as pltpu
from jax.experimental.pallas import tpu_sc as plsc

_K = 16
_MAXF = 1024.0
_NC = 2
_NS = 16
_NW = _NC * _NS
_LANES = 16
_IL = 2


def _topk_sc(rows, lo, hi):
    _, F = rows.shape
    R = hi - lo
    assert R % 8 == 0 and lo % 8 == 0
    ngroups = R // 8
    gpw = ngroups // _NW
    extra = ngroups % _NW
    nb_max = gpw + (1 if extra else 0)
    nchunk = F // _LANES
    RB = 8

    mesh = plsc.VectorSubcoreMesh(core_axis_name="c", subcore_axis_name="s")

    @functools.partial(
        pl.kernel,
        out_type=jax.ShapeDtypeStruct((R, 2 * _K), jnp.float32),
        mesh=mesh,
        scratch_types=[
            pltpu.VMEM((2, RB, F), jnp.float32),
            pltpu.VMEM((nb_max * RB, 2 * _K), jnp.float32),
            pltpu.SemaphoreType.DMA,
            pltpu.SemaphoreType.DMA,
        ],
        compiler_params=pltpu.CompilerParams(needs_layout_passes=False),
    )
    def topk_kernel(w_hbm, cf_out, rbuf, obuf, sem0, sem1):
        wid = lax.axis_index("s") * _NC + lax.axis_index("c")
        nbatch = gpw + jnp.where(wid < extra, 1, 0)
        base = RB * (wid * gpw + jnp.minimum(wid, extra))
        iota = lax.iota(jnp.int32, _LANES)

        def fetch(b, slot, sem):
            return pltpu.make_async_copy(
                w_hbm.at[pl.ds(lo + base + b * RB, RB)], rbuf.at[slot], sem)

        fetch(0, 0, sem0).start()

        def batch_body(b, carry):
            slot = lax.rem(b, 2)

            def even_case():
                fetch(b, 0, sem0).wait()

                @pl.when(b + 1 < nbatch)
                def _():
                    fetch(b + 1, 1, sem1).start()

            def odd_case():
                fetch(b, 1, sem1).wait()

                @pl.when(b + 1 < nbatch)
                def _():
                    fetch(b + 1, 0, sem0).start()

            lax.cond(slot == 0, even_case, odd_case)

            def row_group(rg, carry2):
                for q in range(_IL):
                    r = rg * _IL + q
                    level = []
                    for c in range(nchunk):
                        a = jnp.abs(rbuf[slot, r, pl.ds(c * _LANES, _LANES)])
                        level.append(plsc.sort_key_val(
                            a, iota + c * _LANES, descending=(c % 2 == 1)))
                    while len(level) > 1:
                        nxt = []
                        for m in range(len(level) // 2):
                            av, ai = level[2 * m]
                            bv, bi = level[2 * m + 1]
                            take = av >= bv
                            mv = jnp.where(take, av, bv)
                            mi = jnp.where(take, ai, bi)
                            nxt.append(plsc.sort_key_val(
                                mv, mi, descending=(m % 2 == 1)))
                        level = nxt
                    rv, ri = level[0]
                    slot_v = jnp.full((_LANES,), slot, jnp.int32)
                    row_v = jnp.full((_LANES,), r, jnp.int32)
                    signed = plsc.load_gather(rbuf, [slot_v, row_v, ri])
                    out_r = b * RB + r
                    obuf[out_r, pl.ds(0, _K)] = signed
                    obuf[out_r, pl.ds(_K, _K)] = (
                        ri.astype(jnp.float32) * (_MAXF / F))
                return carry2

            lax.fori_loop(0, RB // _IL, row_group, 0)
            return carry

        lax.fori_loop(0, nbatch, batch_body, 0)

        def flush_extra():
            pltpu.sync_copy(obuf.at[pl.ds(0, nb_max * RB)],
                            cf_out.at[pl.ds(base, nb_max * RB)])

        def flush_base():
            pltpu.sync_copy(obuf.at[pl.ds(0, gpw * RB)],
                            cf_out.at[pl.ds(base, gpw * RB)])

        if extra:
            lax.cond(wid < extra, flush_extra, flush_base)
        else:
            flush_base()

    return topk_kernel(rows)


_C2P = (
    0.9999994436793985,
    -19.73903437293113,
    64.9306133699045,
    -85.29597096153829,
    58.91255532441485,
    -21.28302159300549,
)


def _cos2pi(r):
    u = r - jnp.round(r)
    s = u * u
    p = jnp.float32(_C2P[5])
    for c in (_C2P[4], _C2P[3], _C2P[2], _C2P[1], _C2P[0]):
        p = p * s + jnp.float32(c)
    return p


def _render_body(x_ref, p_ref, w_ref, c3_ref, out_ref):
    i = pl.program_id(1)
    px = p_ref[0:1, :]
    py = p_ref[1:2, :]
    sx = p_ref[2:3, :]
    sy = p_ref[3:4, :]
    rot = p_ref[4:5, :]
    cr = jnp.cos(rot)
    sr = jnp.sin(rot)
    x0 = x_ref[:, 0:1]
    x1 = x_ref[:, 1:2]
    rx = x0 - px
    ry = x1 - py
    tx = cr * rx + sr * ry
    ty = cr * ry - sr * rx
    env = jnp.exp(-0.5 * ((tx * sx) ** 2 + (ty * sy) ** 2))

    wx = w_ref[0, 0:1, :] * _cos2pi(w_ref[0, 2:3, :] * tx)
    wy = w_ref[0, 1:2, :] * _cos2pi(w_ref[0, 3:4, :] * ty)
    for j in range(1, _K):
        wx = wx + w_ref[j, 0:1, :] * _cos2pi(w_ref[j, 2:3, :] * tx)
        wy = wy + w_ref[j, 1:2, :] * _cos2pi(w_ref[j, 3:4, :] * ty)
    w = env * wx * wy
    acc = lax.dot_general(w, c3_ref[...], (((1,), (0,)), ((), ())),
                          preferred_element_type=jnp.float32)

    @pl.when(i == 0)
    def _():
        out_ref[...] = jnp.zeros_like(out_ref)

    out_ref[...] += acc


def _render_half(x, colors, positions, scales, rotations, cfr):
    N = x.shape[0]
    G = colors.shape[0]
    cf = cfr[:, :_K].reshape(G, 2, _K)
    fr = cfr[:, _K:].reshape(G, 2, _K)

    GBLK = 256
    Gp = ((G + GBLK - 1) // GBLK) * GBLK
    pad = Gp - G

    W = jnp.stack([cf[:, 0, :], cf[:, 1, :], fr[:, 0, :], fr[:, 1, :]],
                  axis=1)
    W = jnp.pad(W, ((0, pad), (0, 4), (0, 0))).transpose(2, 1, 0)
    P = jnp.concatenate([positions, scales, rotations,
                         jnp.zeros((G, 3), jnp.float32)], axis=1).T
    P = jnp.pad(P, ((0, 0), (0, pad)))
    C3 = jnp.pad(colors, ((0, pad), (0, 0)))

    NBLK = 128
    return pl.pallas_call(
        _render_body,
        grid=(N // NBLK, Gp // GBLK),
        in_specs=[
            pl.BlockSpec((NBLK, 2), lambda n, i: (n, 0)),
            pl.BlockSpec((8, GBLK), lambda n, i: (0, i)),
            pl.BlockSpec((_K, 8, GBLK), lambda n, i: (0, 0, i)),
            pl.BlockSpec((GBLK, 3), lambda n, i: (i, 0)),
        ],
        out_specs=pl.BlockSpec((NBLK, 3), lambda n, i: (n, 0)),
        out_shape=jax.ShapeDtypeStruct((N, 3), jnp.float32),
    )(x, P, W, C3)


def kernel(x, gaussian_colors, gaussian_positions, gaussian_scales,
           gaussian_rotations, wave_coefficients):
    G, _, F = wave_coefficients.shape
    rows = wave_coefficients.reshape(G * 2, F)

    H = (G // 2) // 4 * 4
    outs = []
    for glo, ghi in ((0, H), (H, G)):
        cfr = _topk_sc(rows, 2 * glo, 2 * ghi)
        outs.append(_render_half(
            x, gaussian_colors[glo:ghi], gaussian_positions[glo:ghi],
            gaussian_scales[glo:ghi], gaussian_rotations[glo:ghi], cfr))
    return outs[0] + outs[1]
